# baseline (speedup 1.0000x reference)
import functools

import jax
import jax.numpy as jnp
from jax import lax
from jax.experimental import pallas as pl
from jax.experimental.pallas import tpu as pltpu
from jax.experimental.pallas import tpu_sc as plsc

TOPK = 8
B, L, C = 256, 256, 32
N = 8192
CDB = C + 1
QBLK = 32
GRID = B // QBLK


def _gelu_exact(v):
    return v * 0.5 * (1.0 + lax.erf(v * 0.7071067811865476))


_OFFS = (-4, -2, -1, 0, 1, 2, 4)


def _encoder_topk_body(x_ref, woff_ref, bias_ref, wf_ref, ax_ref,
                       scores_ref, idx_ref):
    xv = x_ref[...]

    def sh(a, o):
        if o > 0:
            return jnp.concatenate(
                [a[:, o:, :], jnp.zeros((QBLK, o, 64), jnp.float32)], axis=1)
        if o < 0:
            return jnp.concatenate(
                [jnp.zeros((QBLK, -o, 64), jnp.float32), a[:, :o, :]], axis=1)
        return a

    feat = jnp.zeros((QBLK, L, 64), jnp.float32)
    for i, o in enumerate(_OFFS):
        t = jnp.dot(xv, woff_ref[i], preferred_element_type=jnp.float32)
        feat = feat + sh(t.reshape(QBLK, L, 64), o)
    feat = feat + bias_ref[...][None, :, :]
    fv = jnp.sum(_gelu_exact(feat), axis=1) * (1.0 / L)

    out = jnp.dot(fv, wf_ref[...].T, preferred_element_type=jnp.float32)
    mean = jnp.mean(out, axis=1, keepdims=True)
    var = jnp.mean((out - mean) ** 2, axis=1, keepdims=True)
    out = (out - mean) * lax.rsqrt(var + 1e-5)
    nrm = jnp.sqrt(jnp.sum(out * out, axis=1, keepdims=True))
    bx = out / jnp.maximum(nrm, 1e-12)

    s = jax.lax.dot_general(bx, ax_ref[...], (((1,), (1,)), ((), ())),
                            preferred_element_type=jnp.float32)
    iota = lax.broadcasted_iota(jnp.int32, (QBLK, N), 1)
    svals, sidxs = [], []
    for _ in range(TOPK):
        m = jnp.max(s, axis=1, keepdims=True)
        hit = s == m
        idx = jnp.min(jnp.where(hit, iota, jnp.int32(N)), axis=1,
                      keepdims=True)
        svals.append(m)
        sidxs.append(idx)
        s = jnp.where(iota == idx, -jnp.inf, s)
    scores_ref[...] = jnp.concatenate(svals, axis=1)
    idx_ref[...] = jnp.concatenate(sidxs, axis=1)


def _encode_and_topk(x2d, woff, bias, wf, ax):
    return pl.pallas_call(
        _encoder_topk_body,
        grid=(GRID,),
        in_specs=[
            pl.BlockSpec((QBLK * L, C), lambda i: (i, 0)),
            pl.BlockSpec((7, C, 64), lambda i: (0, 0, 0)),
            pl.BlockSpec((1, 64), lambda i: (0, 0)),
            pl.BlockSpec((64, 64), lambda i: (0, 0)),
            pl.BlockSpec((N, 64), lambda i: (0, 0)),
        ],
        out_specs=[
            pl.BlockSpec((QBLK, TOPK), lambda i: (i, 0)),
            pl.BlockSpec((QBLK, TOPK), lambda i: (i, 0)),
        ],
        out_shape=[
            jax.ShapeDtypeStruct((B, TOPK), jnp.float32),
            jax.ShapeDtypeStruct((B, TOPK), jnp.int32),
        ],
    )(x2d, woff, bias, wf, ax)


NSEL = B * TOPK
NWORK = 32
WPW = NSEL // NWORK
NWIN = 4
NCHUNK = WPW // NWIN


_HALves = ((0, 16), (16, 17))
QPW = B // NWORK


def _sc_interleave_body(cmp_hbm, out_hbm,
                        inb0, inb1, inb2, g0, g1, g2, o0, o1, o2):
    wid = lax.axis_index("s") * 2 + lax.axis_index("c")
    qbase = wid * QPW
    inbs = (inb0, inb1, inb2)
    gsems = (g0, g1, g2)
    osems = (o0, o1, o2)

    def fetch(ch, b):
        q, h = ch // 2, ch % 2
        c0, cn = _HALves[h]
        pltpu.async_copy(
            cmp_hbm.at[pl.ds(c0, cn), pl.ds((qbase + q) * TOPK, TOPK), :],
            inbs[b].at[pl.ds(0, cn)], gsems[b])

    def gwait(ch, b):
        cn = _HALves[ch % 2][1]
        pltpu.make_async_copy(cmp_hbm.at[pl.ds(0, cn), pl.ds(0, TOPK), :],
                              inbs[b].at[pl.ds(0, cn)], gsems[b]).wait()

    def flush(ch, b):
        q, h = ch // 2, ch % 2
        c0, cn = _HALves[h]
        pltpu.async_copy(inbs[b].at[pl.ds(0, cn)],
                         out_hbm.at[qbase + q, pl.ds(c0, cn), :, :],
                         osems[b])

    def owait(ch, b):
        c0, cn = _HALves[ch % 2]
        pltpu.make_async_copy(inbs[b].at[pl.ds(0, cn)],
                              out_hbm.at[qbase, pl.ds(c0, cn), :, :],
                              osems[b]).wait()

    fetch(0, 0)
    fetch(1, 1)
    for ch in range(NCHUNK):
        b = ch % 3
        gwait(ch, b)
        flush(ch, b)
        nx = ch + 2
        if nx < NCHUNK:
            nb = nx % 3
            if ch >= 1:
                owait(ch - 1, nb)
            fetch(nx, nb)
    owait(NCHUNK - 2, (NCHUNK - 2) % 3)
    owait(NCHUNK - 1, (NCHUNK - 1) % 3)


def _sc_interleave(compact_t):
    mesh = plsc.VectorSubcoreMesh(core_axis_name="c", subcore_axis_name="s",
                                  num_cores=2, num_subcores=16)
    return pl.kernel(
        _sc_interleave_body,
        out_type=jax.ShapeDtypeStruct((B, CDB, TOPK, L), jnp.float32),
        mesh=mesh,
        scratch_types=[
            pltpu.VMEM((17, TOPK, L), jnp.float32),
            pltpu.VMEM((17, TOPK, L), jnp.float32),
            pltpu.VMEM((17, TOPK, L), jnp.float32),
            pltpu.SemaphoreType.DMA,
            pltpu.SemaphoreType.DMA,
            pltpu.SemaphoreType.DMA,
            pltpu.SemaphoreType.DMA,
            pltpu.SemaphoreType.DMA,
            pltpu.SemaphoreType.DMA,
        ],
        compiler_params=pltpu.CompilerParams(needs_layout_passes=False),
    )(compact_t)


_GLAG = 24


def _tc_gather_body(idx_ref, wc_ref, out_ref, sem):
    def start(j):
        s = idx_ref[j]
        pltpu.make_async_copy(wc_ref.at[:, pl.ds(s, 1), :],
                              out_ref.at[:, pl.ds(j, 1), :], sem).start()

    def drain(j):
        pltpu.make_async_copy(wc_ref.at[:, pl.ds(0, 1), :],
                              out_ref.at[:, pl.ds(j, 1), :], sem).wait()

    def body(j, c):
        start(j)

        @pl.when(j >= _GLAG)
        def _():
            drain(j - _GLAG)
        return c

    lax.fori_loop(0, NSEL, body, 0)

    def tail(j, c):
        drain(NSEL - _GLAG + j)
        return c

    lax.fori_loop(0, _GLAG, tail, 0)


def _tc_gather(wc_t, idx_flat):
    grid_spec = pltpu.PrefetchScalarGridSpec(
        num_scalar_prefetch=1,
        grid=(1,),
        in_specs=[pl.BlockSpec(memory_space=pl.ANY)],
        out_specs=pl.BlockSpec(memory_space=pl.ANY),
        scratch_shapes=[pltpu.SemaphoreType.DMA],
    )
    return pl.pallas_call(
        _tc_gather_body,
        grid_spec=grid_spec,
        out_shape=jax.ShapeDtypeStruct((CDB, NSEL, L), jnp.float32),
    )(idx_flat, wc_t)


def _gather_windows(windows_flat, idx_flat):
    wc_t = jnp.transpose(windows_flat, (1, 0, 2))
    compact_t = _tc_gather(wc_t, idx_flat)
    return jnp.transpose(_sc_interleave(compact_t), (0, 2, 3, 1))


def kernel(x, w1, b1, w2, b2, w3, b3, w4, b4, wf, ax_cache, windows_cache):
    z = jnp.zeros((32, 16), jnp.float32)
    w2t = [w2[:, :, k].T for k in range(3)]
    w3t = [w3[:, :, k].T for k in range(3)]
    w4t = [w4[:, :, k].T for k in range(3)]
    cat = lambda a, b, c, d: jnp.concatenate([a, b, c, d], axis=1)
    woff = jnp.stack([
        cat(z, z, z, w4t[0]),
        cat(z, z, w3t[0], z),
        cat(z, w2t[0], z, z),
        cat(w1[:, :, 0].T, w2t[1], w3t[1], w4t[1]),
        cat(z, w2t[2], z, z),
        cat(z, z, w3t[2], z),
        cat(z, z, z, w4t[2]),
    ])
    bias = jnp.concatenate([b1, b2, b3, b4], axis=0).reshape(1, 64)

    topk_scores, topk_idx = _encode_and_topk(
        x.reshape(B * L, C), woff, bias, wf, ax_cache)

    idx_flat = topk_idx.reshape(B * TOPK)
    windows_raw = _gather_windows(windows_cache, idx_flat)
    return (topk_scores, windows_raw)


# speedup vs baseline: 10.2343x; 10.2343x over previous
import functools

import jax
import jax.numpy as jnp
from jax import lax
from jax.experimental import pallas as pl
from jax.experimental.pallas import tpu as pltpu
from jax.experimental.pallas import tpu_sc as plsc

TOPK = 8
B, L, C = 256, 256, 32
N = 8192
CDB = C + 1
QBLK = 32
GRID = B // QBLK


def _gelu_exact(v):
    return v * 0.5 * (1.0 + lax.erf(v * 0.7071067811865476))


_OFFS = (-4, -2, -1, 0, 1, 2, 4)


def _encoder_topk_body(x_ref, woff_ref, bias_ref, wf_ref, ax_ref,
                       scores_ref, idx_ref):
    xv = x_ref[...]

    def sh(a, o):
        if o > 0:
            return jnp.concatenate(
                [a[:, o:, :], jnp.zeros((QBLK, o, 64), jnp.float32)], axis=1)
        if o < 0:
            return jnp.concatenate(
                [jnp.zeros((QBLK, -o, 64), jnp.float32), a[:, :o, :]], axis=1)
        return a

    feat = jnp.zeros((QBLK, L, 64), jnp.float32)
    for i, o in enumerate(_OFFS):
        t = jnp.dot(xv, woff_ref[i], preferred_element_type=jnp.float32)
        feat = feat + sh(t.reshape(QBLK, L, 64), o)
    feat = feat + bias_ref[...][None, :, :]
    fv = jnp.sum(_gelu_exact(feat), axis=1) * (1.0 / L)

    out = jnp.dot(fv, wf_ref[...].T, preferred_element_type=jnp.float32)
    mean = jnp.mean(out, axis=1, keepdims=True)
    var = jnp.mean((out - mean) ** 2, axis=1, keepdims=True)
    out = (out - mean) * lax.rsqrt(var + 1e-5)
    nrm = jnp.sqrt(jnp.sum(out * out, axis=1, keepdims=True))
    bx = out / jnp.maximum(nrm, 1e-12)

    s = jax.lax.dot_general(bx, ax_ref[...], (((1,), (1,)), ((), ())),
                            preferred_element_type=jnp.float32)
    iota = lax.broadcasted_iota(jnp.int32, (QBLK, N), 1)
    svals, sidxs = [], []
    for _ in range(TOPK):
        m = jnp.max(s, axis=1, keepdims=True)
        hit = s == m
        idx = jnp.min(jnp.where(hit, iota, jnp.int32(N)), axis=1,
                      keepdims=True)
        svals.append(m)
        sidxs.append(idx)
        s = jnp.where(iota == idx, -jnp.inf, s)
    scores_ref[...] = jnp.concatenate(svals, axis=1)
    idx_ref[...] = jnp.concatenate(sidxs, axis=1)


def _encode_and_topk(x2d, woff, bias, wf, ax):
    return pl.pallas_call(
        _encoder_topk_body,
        grid=(GRID,),
        in_specs=[
            pl.BlockSpec((QBLK * L, C), lambda i: (i, 0)),
            pl.BlockSpec((7, C, 64), lambda i: (0, 0, 0)),
            pl.BlockSpec((1, 64), lambda i: (0, 0)),
            pl.BlockSpec((64, 64), lambda i: (0, 0)),
            pl.BlockSpec((N, 64), lambda i: (0, 0)),
        ],
        out_specs=[
            pl.BlockSpec((QBLK, TOPK), lambda i: (i, 0)),
            pl.BlockSpec((QBLK, TOPK), lambda i: (i, 0)),
        ],
        out_shape=[
            jax.ShapeDtypeStruct((B, TOPK), jnp.float32),
            jax.ShapeDtypeStruct((B, TOPK), jnp.int32),
        ],
    )(x2d, woff, bias, wf, ax)


NSEL = B * TOPK
NWORK = 32
WPW = NSEL // NWORK
NWIN = 4
NCHUNK = WPW // NWIN


_HALves = ((0, 16), (16, 17))
QPW = B // NWORK


def _sc_gather_body(wc_hbm, idx_hbm, out_hbm, idx_v,
                    inb0, inb1, inb2, g0, g1, g2, o0, o1, o2):
    wid = lax.axis_index("s") * 2 + lax.axis_index("c")
    qbase = wid * QPW
    pltpu.sync_copy(idx_hbm.at[wid], idx_v)
    lane = lax.iota(jnp.int32, 16)
    inbs = (inb0, inb1, inb2)
    gsems = (g0, g1, g2)
    osems = (o0, o1, o2)

    def fetch(ch, b):
        for w in range(NWIN):
            j = ch * NWIN + w
            v = idx_v[pl.ds((j // 16) * 16, 16)]
            sel = jnp.where(lane == (j % 16), v, 0)
            s = lax.reduce_max(sel, (0,))
            pltpu.async_copy(wc_hbm.at[:, pl.ds(s, 1), :],
                             inbs[b].at[:, pl.ds(w, 1), :], gsems[b])

    def gwait(b):
        pltpu.make_async_copy(wc_hbm.at[:, pl.ds(0, NWIN), :],
                              inbs[b], gsems[b]).wait()

    def flush(ch, b):
        q, k0 = ch // 2, (ch % 2) * NWIN
        pltpu.async_copy(inbs[b],
                         out_hbm.at[qbase + q, :, pl.ds(k0, NWIN), :],
                         osems[b])

    def owait(b):
        pltpu.make_async_copy(inbs[b],
                              out_hbm.at[qbase, :, pl.ds(0, NWIN), :],
                              osems[b]).wait()

    fetch(0, 0)
    fetch(1, 1)
    for ch in range(NCHUNK):
        b = ch % 3
        gwait(b)
        flush(ch, b)
        nx = ch + 2
        if nx < NCHUNK:
            nb = nx % 3
            if ch >= 1:
                owait(nb)
            fetch(nx, nb)
    owait((NCHUNK - 2) % 3)
    owait((NCHUNK - 1) % 3)


def _sc_gather(wc_t, idx):
    mesh = plsc.VectorSubcoreMesh(core_axis_name="c", subcore_axis_name="s",
                                  num_cores=2, num_subcores=16)
    return pl.kernel(
        _sc_gather_body,
        out_type=jax.ShapeDtypeStruct((B, CDB, TOPK, L), jnp.float32),
        mesh=mesh,
        scratch_types=[
            pltpu.VMEM((WPW,), jnp.int32),
            pltpu.VMEM((CDB, NWIN, L), jnp.float32),
            pltpu.VMEM((CDB, NWIN, L), jnp.float32),
            pltpu.VMEM((CDB, NWIN, L), jnp.float32),
            pltpu.SemaphoreType.DMA,
            pltpu.SemaphoreType.DMA,
            pltpu.SemaphoreType.DMA,
            pltpu.SemaphoreType.DMA,
            pltpu.SemaphoreType.DMA,
            pltpu.SemaphoreType.DMA,
        ],
        compiler_params=pltpu.CompilerParams(needs_layout_passes=False),
    )(wc_t, idx.reshape(NWORK, WPW))


def _gather_windows(windows_flat, idx_flat):
    wc_t = jnp.transpose(windows_flat, (1, 0, 2))
    return jnp.transpose(_sc_gather(wc_t, idx_flat), (0, 2, 3, 1))


def kernel(x, w1, b1, w2, b2, w3, b3, w4, b4, wf, ax_cache, windows_cache):
    z = jnp.zeros((32, 16), jnp.float32)
    w2t = [w2[:, :, k].T for k in range(3)]
    w3t = [w3[:, :, k].T for k in range(3)]
    w4t = [w4[:, :, k].T for k in range(3)]
    cat = lambda a, b, c, d: jnp.concatenate([a, b, c, d], axis=1)
    woff = jnp.stack([
        cat(z, z, z, w4t[0]),
        cat(z, z, w3t[0], z),
        cat(z, w2t[0], z, z),
        cat(w1[:, :, 0].T, w2t[1], w3t[1], w4t[1]),
        cat(z, w2t[2], z, z),
        cat(z, z, w3t[2], z),
        cat(z, z, z, w4t[2]),
    ])
    bias = jnp.concatenate([b1, b2, b3, b4], axis=0).reshape(1, 64)

    topk_scores, topk_idx = _encode_and_topk(
        x.reshape(B * L, C), woff, bias, wf, ax_cache)

    idx_flat = topk_idx.reshape(B * TOPK)
    windows_raw = _gather_windows(windows_cache, idx_flat)
    return (topk_scores, windows_raw)


# speedup vs baseline: 10.4233x; 1.0185x over previous
import functools

import jax
import jax.numpy as jnp
from jax import lax
from jax.experimental import pallas as pl
from jax.experimental.pallas import tpu as pltpu
from jax.experimental.pallas import tpu_sc as plsc

TOPK = 8
B, L, C = 256, 256, 32
N = 8192
CDB = C + 1
QBLK = 32
GRID = B // QBLK


def _gelu_exact(v):
    return v * 0.5 * (1.0 + lax.erf(v * 0.7071067811865476))


_OFFS = (-4, -2, -1, 0, 1, 2, 4)


def _encoder_topk_body(x_ref, woff_ref, bias_ref, wf_ref, ax_ref,
                       scores_ref, idx_ref):
    xv = x_ref[...]

    def sh(a, o):
        if o > 0:
            return jnp.concatenate(
                [a[:, o:, :], jnp.zeros((QBLK, o, 64), jnp.float32)], axis=1)
        if o < 0:
            return jnp.concatenate(
                [jnp.zeros((QBLK, -o, 64), jnp.float32), a[:, :o, :]], axis=1)
        return a

    tall = jnp.dot(xv, woff_ref[...], preferred_element_type=jnp.float32)
    tall = tall.reshape(QBLK, L, 7 * 128)
    feat = jnp.zeros((QBLK, L, 64), jnp.float32)
    for i, o in enumerate(_OFFS):
        feat = feat + sh(tall[:, :, i * 128:i * 128 + 64], o)
    feat = feat + bias_ref[...][None, :, :]
    fv = jnp.sum(_gelu_exact(feat), axis=1) * (1.0 / L)

    out = jnp.dot(fv, wf_ref[...].T, preferred_element_type=jnp.float32)
    mean = jnp.mean(out, axis=1, keepdims=True)
    var = jnp.mean((out - mean) ** 2, axis=1, keepdims=True)
    out = (out - mean) * lax.rsqrt(var + 1e-5)
    nrm = jnp.sqrt(jnp.sum(out * out, axis=1, keepdims=True))
    bx = out / jnp.maximum(nrm, 1e-12)

    s = jax.lax.dot_general(bx, ax_ref[...], (((1,), (1,)), ((), ())),
                            preferred_element_type=jnp.float32)
    iota = lax.broadcasted_iota(jnp.int32, (QBLK, N), 1)
    svals, sidxs = [], []
    for _ in range(TOPK):
        m = jnp.max(s, axis=1, keepdims=True)
        hit = s == m
        idx = jnp.min(jnp.where(hit, iota, jnp.int32(N)), axis=1,
                      keepdims=True)
        svals.append(m)
        sidxs.append(idx)
        s = jnp.where(iota == idx, -jnp.inf, s)
    scores_ref[...] = jnp.concatenate(svals, axis=1)
    idx_ref[...] = jnp.concatenate(sidxs, axis=1)


def _encode_and_topk(x2d, woff, bias, wf, ax):
    return pl.pallas_call(
        _encoder_topk_body,
        grid=(GRID,),
        in_specs=[
            pl.BlockSpec((QBLK * L, C), lambda i: (i, 0)),
            pl.BlockSpec((C, 7 * 128), lambda i: (0, 0)),
            pl.BlockSpec((1, 64), lambda i: (0, 0)),
            pl.BlockSpec((64, 64), lambda i: (0, 0)),
            pl.BlockSpec((N, 64), lambda i: (0, 0)),
        ],
        out_specs=[
            pl.BlockSpec((QBLK, TOPK), lambda i: (i, 0)),
            pl.BlockSpec((QBLK, TOPK), lambda i: (i, 0)),
        ],
        out_shape=[
            jax.ShapeDtypeStruct((B, TOPK), jnp.float32),
            jax.ShapeDtypeStruct((B, TOPK), jnp.int32),
        ],
    )(x2d, woff, bias, wf, ax)


NSEL = B * TOPK
NWORK = 32
WPW = NSEL // NWORK
NWIN = 4
NCHUNK = WPW // NWIN


_HALves = ((0, 16), (16, 17))
QPW = B // NWORK


def _sc_gather_body(wc_hbm, idx_hbm, out_hbm, idx_v,
                    inb0, inb1, inb2, g0, g1, g2, o0, o1, o2):
    wid = lax.axis_index("s") * 2 + lax.axis_index("c")
    qbase = wid * QPW
    pltpu.sync_copy(idx_hbm.at[wid], idx_v)
    lane = lax.iota(jnp.int32, 16)
    inbs = (inb0, inb1, inb2)
    gsems = (g0, g1, g2)
    osems = (o0, o1, o2)

    def fetch(ch, b):
        for w in range(NWIN):
            j = ch * NWIN + w
            v = idx_v[pl.ds((j // 16) * 16, 16)]
            sel = jnp.where(lane == (j % 16), v, 0)
            s = lax.reduce_max(sel, (0,))
            pltpu.async_copy(wc_hbm.at[:, pl.ds(s, 1), :],
                             inbs[b].at[:, pl.ds(w, 1), :], gsems[b])

    def gwait(b):
        pltpu.make_async_copy(wc_hbm.at[:, pl.ds(0, NWIN), :],
                              inbs[b], gsems[b]).wait()

    def flush(ch, b):
        q, k0 = ch // 2, (ch % 2) * NWIN
        pltpu.async_copy(inbs[b],
                         out_hbm.at[qbase + q, :, pl.ds(k0, NWIN), :],
                         osems[b])

    def owait(b):
        pltpu.make_async_copy(inbs[b],
                              out_hbm.at[qbase, :, pl.ds(0, NWIN), :],
                              osems[b]).wait()

    fetch(0, 0)
    fetch(1, 1)
    for ch in range(NCHUNK):
        b = ch % 3
        gwait(b)
        flush(ch, b)
        nx = ch + 2
        if nx < NCHUNK:
            nb = nx % 3
            if ch >= 1:
                owait(nb)
            fetch(nx, nb)
    owait((NCHUNK - 2) % 3)
    owait((NCHUNK - 1) % 3)


def _sc_gather(wc_t, idx):
    mesh = plsc.VectorSubcoreMesh(core_axis_name="c", subcore_axis_name="s",
                                  num_cores=2, num_subcores=16)
    return pl.kernel(
        _sc_gather_body,
        out_type=jax.ShapeDtypeStruct((B, CDB, TOPK, L), jnp.float32),
        mesh=mesh,
        scratch_types=[
            pltpu.VMEM((WPW,), jnp.int32),
            pltpu.VMEM((CDB, NWIN, L), jnp.float32),
            pltpu.VMEM((CDB, NWIN, L), jnp.float32),
            pltpu.VMEM((CDB, NWIN, L), jnp.float32),
            pltpu.SemaphoreType.DMA,
            pltpu.SemaphoreType.DMA,
            pltpu.SemaphoreType.DMA,
            pltpu.SemaphoreType.DMA,
            pltpu.SemaphoreType.DMA,
            pltpu.SemaphoreType.DMA,
        ],
        compiler_params=pltpu.CompilerParams(needs_layout_passes=False),
    )(wc_t, idx.reshape(NWORK, WPW))


def _gather_windows(windows_flat, idx_flat):
    wc_t = jnp.transpose(windows_flat, (1, 0, 2))
    return jnp.transpose(_sc_gather(wc_t, idx_flat), (0, 2, 3, 1))


def kernel(x, w1, b1, w2, b2, w3, b3, w4, b4, wf, ax_cache, windows_cache):
    z = jnp.zeros((32, 16), jnp.float32)
    w2t = [w2[:, :, k].T for k in range(3)]
    w3t = [w3[:, :, k].T for k in range(3)]
    w4t = [w4[:, :, k].T for k in range(3)]
    pad = jnp.zeros((32, 64), jnp.float32)
    cat = lambda a, b, c, d: jnp.concatenate([a, b, c, d, pad], axis=1)
    woff = jnp.concatenate([
        cat(z, z, z, w4t[0]),
        cat(z, z, w3t[0], z),
        cat(z, w2t[0], z, z),
        cat(w1[:, :, 0].T, w2t[1], w3t[1], w4t[1]),
        cat(z, w2t[2], z, z),
        cat(z, z, w3t[2], z),
        cat(z, z, z, w4t[2]),
    ], axis=1)
    bias = jnp.concatenate([b1, b2, b3, b4], axis=0).reshape(1, 64)

    topk_scores, topk_idx = _encode_and_topk(
        x.reshape(B * L, C), woff, bias, wf, ax_cache)

    idx_flat = topk_idx.reshape(B * TOPK)
    windows_raw = _gather_windows(windows_cache, idx_flat)
    return (topk_scores, windows_raw)
